# Initial kernel scaffold; baseline (speedup 1.0000x reference)
import jax
import jax.numpy as jnp
from jax.experimental import pallas as pl


def kernel(nodes, indices, key_W, key_b, gate_W, gate_b, query_W, query_b, value_W, value_b, out_W, out_b):
    raise NotImplementedError("write your pallas kernel here")


# speedup vs baseline: 3.8494x; 3.8494x over previous
import functools

import jax
import jax.numpy as jnp
from jax import lax
from jax.experimental import pallas as pl
from jax.experimental.pallas import tpu as pltpu
from jax.experimental.pallas import tpu_sc as plsc

N = 320000
D = 128
S = 10000
NT = 32
SEG_T = 320
S_PAD = NT * SEG_T
EXW = 16
ROWS_B = 4000
CHUNK = 80
BIG = 1 << 30


def _stage0_body(idx_ref, th_ref, out_ref):
    i = pl.program_id(0)

    @pl.when(i == 0)
    def _():
        out_ref[...] = jnp.zeros_like(out_ref)

    mask = idx_ref[...] < th_ref[...]
    out_ref[...] += jnp.sum(mask.astype(jnp.int32), axis=0, keepdims=True)


def _stage0(indices):
    nblocks = 40
    b = N // nblocks
    return pl.pallas_call(
        _stage0_body,
        grid=(nblocks,),
        in_specs=[
            pl.BlockSpec((b, 1), lambda i: (i, 0)),
            pl.BlockSpec((1, 128), lambda i: (0, 0)),
        ],
        out_specs=pl.BlockSpec((1, 128), lambda i: (0, 0)),
        out_shape=jax.ShapeDtypeStruct((1, 128), jnp.int32),
    )(indices.reshape(N, 1),
      jnp.where(jnp.arange(128) <= NT, jnp.arange(128) * SEG_T, BIG)
      .astype(jnp.int32).reshape(1, 128))


def _stage1_body(x_ref, kw_ref, kb_ref, gw_ref, gb_ref, qw_ref, qb_ref,
                 scaled_ref, ex_ref):
    x = x_ref[...]
    k = jnp.tanh(jnp.dot(x, kw_ref[...], preferred_element_type=jnp.float32)
                 + kb_ref[...])
    g = jax.nn.sigmoid(
        jnp.dot(x, gw_ref[...], preferred_element_type=jnp.float32)
        + gb_ref[...])
    logit = (jnp.dot(k * g, qw_ref[...], preferred_element_type=jnp.float32)
             + qb_ref[...])
    ex = jnp.exp(logit)
    scaled_ref[...] = x * ex
    ex_ref[...] = jnp.broadcast_to(ex, (x.shape[0], EXW))


def _stage1(nodes, key_W, key_b, gate_W, gate_b, query_W, query_b):
    nblocks = N // ROWS_B
    rep = lambda *shape: pl.BlockSpec(shape, lambda i: (0,) * len(shape))
    return pl.pallas_call(
        _stage1_body,
        grid=(nblocks,),
        in_specs=[
            pl.BlockSpec((ROWS_B, D), lambda i: (i, 0)),
            rep(D, D), rep(1, D), rep(D, 1), rep(1, 1), rep(D, 1), rep(1, 1),
        ],
        out_specs=[
            pl.BlockSpec((ROWS_B, D), lambda i: (i, 0)),
            pl.BlockSpec((ROWS_B, EXW), lambda i: (i, 0)),
        ],
        out_shape=[
            jax.ShapeDtypeStruct((N, D), jnp.float32),
            jax.ShapeDtypeStruct((N, EXW), jnp.float32),
        ],
    )(nodes, key_W, key_b.reshape(1, D), gate_W, gate_b.reshape(1, 1),
      query_W, query_b.reshape(1, 1))


def _zero_vmem1(ref, n):
    def b(i, carry):
        ref[pl.ds(i * 16, 16)] = jnp.zeros((16,), jnp.float32)
        return carry

    lax.fori_loop(0, n // 16, b, 0)


def _vmem_scalar(ref, i):
    out = jnp.int32(0)
    for grp in range(3):
        v = ref[pl.ds(grp * 16, 16)]
        for r in range(16):
            out = jnp.where(i == grp * 16 + r, v[r], out)
    return out


def _stage2(scaled, expad, indices, bounds):
    info = plsc.get_sparse_core_info()
    nc, ns = info.num_cores, info.num_subcores

    mesh = plsc.VectorSubcoreMesh(core_axis_name="c", subcore_axis_name="s")

    @functools.partial(
        pl.kernel,
        out_type=[
            jax.ShapeDtypeStruct((S_PAD * D,), jnp.float32),
            jax.ShapeDtypeStruct((S_PAD * EXW,), jnp.float32),
        ],
        mesh=mesh,
        scratch_types=[
            pltpu.VMEM(((SEG_T + 8) * D,), jnp.float32),
            pltpu.VMEM(((SEG_T + 8) * EXW,), jnp.float32),
            pltpu.VMEM((CHUNK, D), jnp.float32),
            pltpu.VMEM((CHUNK, EXW), jnp.float32),
            pltpu.VMEM((CHUNK,), jnp.int32),
            pltpu.VMEM((128,), jnp.int32),
        ],
    )
    def sc_kernel(scaled_hbm, expad_hbm, idx_hbm, bounds_hbm,
                  pooled_out, denom_out,
                  acc, dacc, rows_v, ex_v, idx_v, bnd_v):
        c = lax.axis_index("c")
        s = lax.axis_index("s")
        t = s * nc + c
        segbase = t * SEG_T
        lanes = lax.iota(jnp.int32, 16)

        _zero_vmem1(acc, (SEG_T + 8) * D)
        _zero_vmem1(dacc, (SEG_T + 8) * EXW)

        pltpu.sync_copy(bounds_hbm, bnd_v)
        start = _vmem_scalar(bnd_v, t)
        end = _vmem_scalar(bnd_v, t + 1)
        start8 = (start // 8) * 8
        end8 = ((end + 7) // 8) * 8
        rows8 = end8 - start8

        def group(goff, gn):
            lv = idx_v[pl.ds(goff, 16)] - segbase
            validv = (lv >= 0) & (lv < SEG_T)
            lid2v = jnp.where(validv, lv, SEG_T)
            for r in range(gn):
                rbase = lid2v[r] * D
                for j in range(D // 16):
                    plsc.addupdate(acc.at[pl.ds(rbase + 16 * j, 16)],
                                   rows_v[goff + r, pl.ds(16 * j, 16)])
                plsc.addupdate(dacc.at[pl.ds(lid2v[r] * EXW, 16)],
                               ex_v[goff + r, pl.ds(0, 16)])

        def chunk(off, nrows):
            pltpu.sync_copy(idx_hbm.at[pl.ds(off, nrows)],
                            idx_v.at[pl.ds(0, nrows)])
            pltpu.sync_copy(scaled_hbm.at[pl.ds(off, nrows)],
                            rows_v.at[pl.ds(0, nrows)])
            pltpu.sync_copy(expad_hbm.at[pl.ds(off, nrows)],
                            ex_v.at[pl.ds(0, nrows)])
            if nrows >= 16:
                for g in range(nrows // 16):
                    group(16 * g, 16)
            else:
                group(0, nrows)

        def cbody(k, carry):
            chunk(start8 + k * CHUNK, CHUNK)
            return carry

        lax.fori_loop(0, rows8 // CHUNK, cbody, 0)

        def tbody(k, carry):
            chunk(start8 + (rows8 // CHUNK) * CHUNK + k * 8, 8)
            return carry

        lax.fori_loop(0, (rows8 % CHUNK) // 8, tbody, 0)

        pltpu.sync_copy(acc.at[pl.ds(0, SEG_T * D)],
                        pooled_out.at[pl.ds(segbase * D, SEG_T * D)])
        pltpu.sync_copy(dacc.at[pl.ds(0, SEG_T * EXW)],
                        denom_out.at[pl.ds(segbase * EXW, SEG_T * EXW)])

    return sc_kernel(scaled, expad, indices, bounds)


def _stage3_body(p_ref, d_ref, vw_ref, vb_ref, ow_ref, ob_ref, out_ref):
    pooled = p_ref[...]
    den = d_ref[:, 0:1]
    safe = den > 0.0
    inv = jnp.where(safe, 1.0 / den, 0.0)
    res = (jnp.dot(pooled * inv, vw_ref[...],
                   preferred_element_type=jnp.float32)
           + safe.astype(jnp.float32) * vb_ref[...])
    out_ref[...] = (jnp.dot(res, ow_ref[...],
                            preferred_element_type=jnp.float32)
                    + ob_ref[...])


def _stage3(pooled, denom, value_W, value_b, out_W, out_b):
    return pl.pallas_call(
        _stage3_body,
        out_shape=jax.ShapeDtypeStruct((S_PAD, D), jnp.float32),
    )(pooled, denom, value_W, value_b.reshape(1, D),
      out_W, out_b.reshape(1, D))


def kernel(nodes, indices, key_W, key_b, gate_W, gate_b, query_W, query_b,
           value_W, value_b, out_W, out_b):
    idx32 = indices.astype(jnp.int32)
    bounds = _stage0(idx32).reshape(128)
    scaled, expad = _stage1(nodes, key_W, key_b, gate_W, gate_b,
                            query_W, query_b)
    pooled, denom = _stage2(scaled, expad, idx32, bounds)
    pooled = pooled.reshape(S_PAD, D)
    denom = denom.reshape(S_PAD, EXW)
    out_pad = _stage3(pooled, denom, value_W, value_b, out_W, out_b)
    return out_pad[:S]


# speedup vs baseline: 4.7844x; 1.2429x over previous
import functools

import jax
import jax.numpy as jnp
from jax import lax
from jax.experimental import pallas as pl
from jax.experimental.pallas import tpu as pltpu
from jax.experimental.pallas import tpu_sc as plsc

N = 320000
D = 128
S = 10000
NT = 32
SEG_T = 320
S_PAD = NT * SEG_T
EXW = 16
ROWS_B = 4000
CHUNK = 160
BIG = 1 << 30


def _stage0_body(idx_ref, th_ref, out_ref):
    i = pl.program_id(0)

    @pl.when(i == 0)
    def _():
        out_ref[...] = jnp.zeros_like(out_ref)

    mask = idx_ref[...] < th_ref[...]
    out_ref[...] += jnp.sum(mask.astype(jnp.int32), axis=0, keepdims=True)


def _stage0(indices):
    nblocks = 40
    b = N // nblocks
    return pl.pallas_call(
        _stage0_body,
        grid=(nblocks,),
        in_specs=[
            pl.BlockSpec((b, 1), lambda i: (i, 0)),
            pl.BlockSpec((1, 128), lambda i: (0, 0)),
        ],
        out_specs=pl.BlockSpec((1, 128), lambda i: (0, 0)),
        out_shape=jax.ShapeDtypeStruct((1, 128), jnp.int32),
    )(indices.reshape(N, 1),
      jnp.where(jnp.arange(128) <= NT, jnp.arange(128) * SEG_T, BIG)
      .astype(jnp.int32).reshape(1, 128))


def _stage1_body(x_ref, kw_ref, kb_ref, gw_ref, gb_ref, qw_ref, qb_ref,
                 scaled_ref, ex_ref):
    x = x_ref[...]
    k = jnp.tanh(jnp.dot(x, kw_ref[...], preferred_element_type=jnp.float32)
                 + kb_ref[...])
    g = jax.nn.sigmoid(
        jnp.dot(x, gw_ref[...], preferred_element_type=jnp.float32)
        + gb_ref[...])
    logit = (jnp.dot(k * g, qw_ref[...], preferred_element_type=jnp.float32)
             + qb_ref[...])
    ex = jnp.exp(logit)
    scaled_ref[...] = x * ex
    ex_ref[...] = jnp.broadcast_to(ex, (x.shape[0], EXW))


def _stage1(nodes, key_W, key_b, gate_W, gate_b, query_W, query_b):
    nblocks = N // ROWS_B
    rep = lambda *shape: pl.BlockSpec(shape, lambda i: (0,) * len(shape))
    return pl.pallas_call(
        _stage1_body,
        grid=(nblocks,),
        in_specs=[
            pl.BlockSpec((ROWS_B, D), lambda i: (i, 0)),
            rep(D, D), rep(1, D), rep(D, 1), rep(1, 1), rep(D, 1), rep(1, 1),
        ],
        out_specs=[
            pl.BlockSpec((ROWS_B, D), lambda i: (i, 0)),
            pl.BlockSpec((ROWS_B, EXW), lambda i: (i, 0)),
        ],
        out_shape=[
            jax.ShapeDtypeStruct((N, D), jnp.float32),
            jax.ShapeDtypeStruct((N, EXW), jnp.float32),
        ],
    )(nodes, key_W, key_b.reshape(1, D), gate_W, gate_b.reshape(1, 1),
      query_W, query_b.reshape(1, 1))


def _zero_vmem1(ref, n):
    def b(i, carry):
        ref[pl.ds(i * 16, 16)] = jnp.zeros((16,), jnp.float32)
        return carry

    lax.fori_loop(0, n // 16, b, 0)


def _vmem_scalar(ref, i):
    out = jnp.int32(0)
    for grp in range(3):
        v = ref[pl.ds(grp * 16, 16)]
        for r in range(16):
            out = jnp.where(i == grp * 16 + r, v[r], out)
    return out


def _stage2(scaled, expad, indices, bounds):
    info = plsc.get_sparse_core_info()
    nc, ns = info.num_cores, info.num_subcores

    mesh = plsc.VectorSubcoreMesh(core_axis_name="c", subcore_axis_name="s")

    @functools.partial(
        pl.kernel,
        out_type=[
            jax.ShapeDtypeStruct((S_PAD * D,), jnp.float32),
            jax.ShapeDtypeStruct((S_PAD * EXW,), jnp.float32),
        ],
        mesh=mesh,
        scratch_types=[
            pltpu.VMEM(((SEG_T + 8) * D,), jnp.float32),
            pltpu.VMEM(((SEG_T + 8) * EXW,), jnp.float32),
            pltpu.VMEM((2 * CHUNK, D), jnp.float32),
            pltpu.VMEM((2 * CHUNK, EXW), jnp.float32),
            pltpu.VMEM((2 * CHUNK,), jnp.int32),
            pltpu.VMEM((128,), jnp.int32),
            pltpu.SemaphoreType.DMA,
            pltpu.SemaphoreType.DMA,
            pltpu.SemaphoreType.DMA,
        ],
    )
    def sc_kernel(scaled_hbm, expad_hbm, idx_hbm, bounds_hbm,
                  pooled_out, denom_out,
                  acc, dacc, rows_v, ex_v, idx_v, bnd_v,
                  rsem, esem, isem):
        c = lax.axis_index("c")
        s = lax.axis_index("s")
        t = s * nc + c
        segbase = t * SEG_T
        lanes = lax.iota(jnp.int32, 16)

        _zero_vmem1(acc, (SEG_T + 8) * D)
        _zero_vmem1(dacc, (SEG_T + 8) * EXW)

        pltpu.sync_copy(bounds_hbm, bnd_v)
        start = _vmem_scalar(bnd_v, t)
        end = _vmem_scalar(bnd_v, t + 1)
        start8 = (start // 8) * 8
        end8 = ((end + 7) // 8) * 8
        rows8 = end8 - start8

        nch = rows8 // CHUNK

        def group(boff, goff, gn):
            lv = idx_v[pl.ds(boff + goff, 16)] - segbase
            validv = (lv >= 0) & (lv < SEG_T)
            lid2v = jnp.where(validv, lv, SEG_T)
            for r in range(gn):
                rbase = lid2v[r] * D
                for j in range(D // 16):
                    plsc.addupdate(acc.at[pl.ds(rbase + 16 * j, 16)],
                                   rows_v[boff + goff + r, pl.ds(16 * j, 16)])
                plsc.addupdate(dacc.at[pl.ds(lid2v[r] * EXW, 16)],
                               ex_v[boff + goff + r, pl.ds(0, 16)])

        def issue(k):
            off = start8 + k * CHUNK
            boff = (k % 2) * CHUNK
            pltpu.async_copy(idx_hbm.at[pl.ds(off, CHUNK)],
                             idx_v.at[pl.ds(boff, CHUNK)], isem)
            pltpu.async_copy(scaled_hbm.at[pl.ds(off, CHUNK)],
                             rows_v.at[pl.ds(boff, CHUNK)], rsem)
            pltpu.async_copy(expad_hbm.at[pl.ds(off, CHUNK)],
                             ex_v.at[pl.ds(boff, CHUNK)], esem)

        @pl.when(nch > 0)
        def _():
            issue(0)

        def cbody(k, carry):
            boff = (k % 2) * CHUNK
            pltpu.make_async_copy(idx_hbm.at[pl.ds(0, CHUNK)],
                                  idx_v.at[pl.ds(boff, CHUNK)], isem).wait()
            pltpu.make_async_copy(scaled_hbm.at[pl.ds(0, CHUNK)],
                                  rows_v.at[pl.ds(boff, CHUNK)], rsem).wait()
            pltpu.make_async_copy(expad_hbm.at[pl.ds(0, CHUNK)],
                                  ex_v.at[pl.ds(boff, CHUNK)], esem).wait()

            @pl.when(k + 1 < nch)
            def _():
                issue(k + 1)

            for g in range(CHUNK // 16):
                group(boff, 16 * g, 16)
            return carry

        lax.fori_loop(0, nch, cbody, 0)

        def tbody(k, carry):
            off = start8 + nch * CHUNK + k * 8
            pltpu.sync_copy(idx_hbm.at[pl.ds(off, 8)],
                            idx_v.at[pl.ds(0, 8)])
            pltpu.sync_copy(scaled_hbm.at[pl.ds(off, 8)],
                            rows_v.at[pl.ds(0, 8)])
            pltpu.sync_copy(expad_hbm.at[pl.ds(off, 8)],
                            ex_v.at[pl.ds(0, 8)])
            group(0, 0, 8)
            return carry

        lax.fori_loop(0, (rows8 % CHUNK) // 8, tbody, 0)

        pltpu.sync_copy(acc.at[pl.ds(0, SEG_T * D)],
                        pooled_out.at[pl.ds(segbase * D, SEG_T * D)])
        pltpu.sync_copy(dacc.at[pl.ds(0, SEG_T * EXW)],
                        denom_out.at[pl.ds(segbase * EXW, SEG_T * EXW)])

    return sc_kernel(scaled, expad, indices, bounds)


def _stage3_body(p_ref, d_ref, vw_ref, vb_ref, ow_ref, ob_ref, out_ref):
    pooled = p_ref[...]
    den = d_ref[:, 0:1]
    safe = den > 0.0
    inv = jnp.where(safe, 1.0 / den, 0.0)
    res = (jnp.dot(pooled * inv, vw_ref[...],
                   preferred_element_type=jnp.float32)
           + safe.astype(jnp.float32) * vb_ref[...])
    out_ref[...] = (jnp.dot(res, ow_ref[...],
                            preferred_element_type=jnp.float32)
                    + ob_ref[...])


def _stage3(pooled, denom, value_W, value_b, out_W, out_b):
    return pl.pallas_call(
        _stage3_body,
        out_shape=jax.ShapeDtypeStruct((S_PAD, D), jnp.float32),
    )(pooled, denom, value_W, value_b.reshape(1, D),
      out_W, out_b.reshape(1, D))


def kernel(nodes, indices, key_W, key_b, gate_W, gate_b, query_W, query_b,
           value_W, value_b, out_W, out_b):
    idx32 = indices.astype(jnp.int32)
    bounds = _stage0(idx32).reshape(128)
    scaled, expad = _stage1(nodes, key_W, key_b, gate_W, gate_b,
                            query_W, query_b)
    pooled, denom = _stage2(scaled, expad, idx32, bounds)
    pooled = pooled.reshape(S_PAD, D)
    denom = denom.reshape(S_PAD, EXW)
    out_pad = _stage3(pooled, denom, value_W, value_b, out_W, out_b)
    return out_pad[:S]


# speedup vs baseline: 6.5330x; 1.3655x over previous
import functools

import jax
import jax.numpy as jnp
from jax import lax
from jax.experimental import pallas as pl
from jax.experimental.pallas import tpu as pltpu
from jax.experimental.pallas import tpu_sc as plsc

N = 320000
D = 128
S = 10000
NT = 32
SEG_T = 320
S_PAD = NT * SEG_T
EXW = 16
ROWS_B = 4000
CHUNK = 96
COLS = D + EXW
BIG = 1 << 30


def _stage0_body(idx_ref, th_ref, out_ref):
    i = pl.program_id(0)

    @pl.when(i == 0)
    def _():
        out_ref[...] = jnp.zeros_like(out_ref)

    mask = idx_ref[...] < th_ref[...]
    out_ref[...] += jnp.sum(mask.astype(jnp.int32), axis=0, keepdims=True)


def _stage0(indices):
    nblocks = 40
    b = N // nblocks
    return pl.pallas_call(
        _stage0_body,
        grid=(nblocks,),
        in_specs=[
            pl.BlockSpec((b, 1), lambda i: (i, 0)),
            pl.BlockSpec((1, 128), lambda i: (0, 0)),
        ],
        out_specs=pl.BlockSpec((1, 128), lambda i: (0, 0)),
        out_shape=jax.ShapeDtypeStruct((1, 128), jnp.int32),
    )(indices.reshape(N, 1),
      jnp.where(jnp.arange(128) <= NT, jnp.arange(128) * SEG_T, BIG)
      .astype(jnp.int32).reshape(1, 128))


def _stage1_body(x_ref, kw_ref, kb_ref, gw_ref, gb_ref, qw_ref, qb_ref,
                 comb_ref):
    x = x_ref[...]
    k = jnp.tanh(jnp.dot(x, kw_ref[...], preferred_element_type=jnp.float32)
                 + kb_ref[...])
    g = jax.nn.sigmoid(
        jnp.dot(x, gw_ref[...], preferred_element_type=jnp.float32)
        + gb_ref[...])
    logit = (jnp.dot(k * g, qw_ref[...], preferred_element_type=jnp.float32)
             + qb_ref[...])
    ex = jnp.exp(logit)
    comb_ref[:, 0:D] = x * ex
    comb_ref[:, D:COLS] = jnp.broadcast_to(ex, (x.shape[0], EXW))


def _stage1(nodes, key_W, key_b, gate_W, gate_b, query_W, query_b):
    nblocks = N // ROWS_B
    rep = lambda *shape: pl.BlockSpec(shape, lambda i: (0,) * len(shape))
    return pl.pallas_call(
        _stage1_body,
        grid=(nblocks,),
        in_specs=[
            pl.BlockSpec((ROWS_B, D), lambda i: (i, 0)),
            rep(D, D), rep(1, D), rep(D, 1), rep(1, 1), rep(D, 1), rep(1, 1),
        ],
        out_specs=pl.BlockSpec((ROWS_B, COLS), lambda i: (i, 0)),
        out_shape=jax.ShapeDtypeStruct((N, COLS), jnp.float32),
    )(nodes, key_W, key_b.reshape(1, D), gate_W, gate_b.reshape(1, 1),
      query_W, query_b.reshape(1, 1))


def _zero_vmem1(ref, n):
    def b(i, carry):
        ref[pl.ds(i * 16, 16)] = jnp.zeros((16,), jnp.float32)
        return carry

    lax.fori_loop(0, n // 16, b, 0)


def _vmem_scalar(ref, i):
    out = jnp.int32(0)
    for grp in range(3):
        v = ref[pl.ds(grp * 16, 16)]
        for r in range(16):
            out = jnp.where(i == grp * 16 + r, v[r], out)
    return out


def _stage2(comb, indices, bounds):
    info = plsc.get_sparse_core_info()
    nc, ns = info.num_cores, info.num_subcores
    nj = COLS // 16

    mesh = plsc.VectorSubcoreMesh(core_axis_name="c", subcore_axis_name="s")

    @functools.partial(
        pl.kernel,
        out_type=jax.ShapeDtypeStruct((S_PAD * COLS,), jnp.float32),
        mesh=mesh,
        scratch_types=[
            pltpu.VMEM(((SEG_T + 8) * COLS,), jnp.float32),
            pltpu.VMEM((2 * CHUNK, COLS), jnp.float32),
            pltpu.VMEM((2 * CHUNK,), jnp.int32),
            pltpu.VMEM((128,), jnp.int32),
            pltpu.SemaphoreType.DMA,
            pltpu.SemaphoreType.DMA,
        ],
    )
    def sc_kernel(comb_hbm, idx_hbm, bounds_hbm, pooled_out,
                  acc, rows_v, idx_v, bnd_v, rsem, isem):
        c = lax.axis_index("c")
        s = lax.axis_index("s")
        t = s * nc + c
        segbase = t * SEG_T

        _zero_vmem1(acc, (SEG_T + 8) * COLS)

        pltpu.sync_copy(bounds_hbm, bnd_v)
        start = _vmem_scalar(bnd_v, t)
        end = _vmem_scalar(bnd_v, t + 1)
        start8 = (start // 8) * 8
        end8 = ((end + 7) // 8) * 8
        rows8 = end8 - start8
        nch = rows8 // CHUNK

        zero16 = jnp.zeros((16,), jnp.float32)

        def group(boff, goff, gn, cur, vacc):
            lv = idx_v[pl.ds(boff + goff, 16)] - segbase
            validv = (lv >= 0) & (lv < SEG_T)
            lid2v = jnp.where(validv, lv, SEG_T)
            for r in range(gn):
                lid = lid2v[r]
                row = [rows_v[boff + goff + r, pl.ds(16 * j, 16)]
                       for j in range(nj)]
                changed = lid != cur

                @pl.when(changed)
                def _(cur=cur, vacc=tuple(vacc)):
                    base = cur * COLS
                    for j in range(nj):
                        plsc.addupdate(acc.at[pl.ds(base + 16 * j, 16)],
                                       vacc[j])

                vacc = [jnp.where(changed, row[j], vacc[j] + row[j])
                        for j in range(nj)]
                cur = jnp.where(changed, lid, cur)
            return cur, vacc

        def issue(k):
            off = start8 + k * CHUNK
            boff = (k % 2) * CHUNK
            pltpu.async_copy(idx_hbm.at[pl.ds(off, CHUNK)],
                             idx_v.at[pl.ds(boff, CHUNK)], isem)
            pltpu.async_copy(comb_hbm.at[pl.ds(off, CHUNK)],
                             rows_v.at[pl.ds(boff, CHUNK)], rsem)

        @pl.when(nch > 0)
        def _():
            issue(0)

        def cbody(k, carry):
            cur, vacc = carry[0], list(carry[1:])
            boff = (k % 2) * CHUNK
            pltpu.make_async_copy(idx_hbm.at[pl.ds(0, CHUNK)],
                                  idx_v.at[pl.ds(boff, CHUNK)], isem).wait()
            pltpu.make_async_copy(comb_hbm.at[pl.ds(0, CHUNK)],
                                  rows_v.at[pl.ds(boff, CHUNK)], rsem).wait()

            @pl.when(k + 1 < nch)
            def _():
                issue(k + 1)

            for g in range(CHUNK // 16):
                cur, vacc = group(boff, 16 * g, 16, cur, vacc)
            return (cur, *vacc)

        init = (jnp.int32(SEG_T),) + (zero16,) * nj
        carry = lax.fori_loop(0, nch, cbody, init)

        def tbody(k, carry):
            cur, vacc = carry[0], list(carry[1:])
            off = start8 + nch * CHUNK + k * 8
            pltpu.sync_copy(idx_hbm.at[pl.ds(off, 8)], idx_v.at[pl.ds(0, 8)])
            pltpu.sync_copy(comb_hbm.at[pl.ds(off, 8)],
                            rows_v.at[pl.ds(0, 8)])
            cur, vacc = group(0, 0, 8, cur, vacc)
            return (cur, *vacc)

        carry = lax.fori_loop(0, (rows8 % CHUNK) // 8, tbody, carry)

        cur, vacc = carry[0], list(carry[1:])
        base = cur * COLS
        for j in range(nj):
            plsc.addupdate(acc.at[pl.ds(base + 16 * j, 16)], vacc[j])

        pltpu.sync_copy(acc.at[pl.ds(0, SEG_T * COLS)],
                        pooled_out.at[pl.ds(segbase * COLS, SEG_T * COLS)])

    return sc_kernel(comb, indices, bounds)


def _stage3_body(p_ref, vw_ref, vb_ref, ow_ref, ob_ref, out_ref):
    pooled = p_ref[:, 0:D]
    den = p_ref[:, D:D + 1]
    safe = den > 0.0
    inv = jnp.where(safe, 1.0 / den, 0.0)
    res = (jnp.dot(pooled * inv, vw_ref[...],
                   preferred_element_type=jnp.float32)
           + safe.astype(jnp.float32) * vb_ref[...])
    out_ref[...] = (jnp.dot(res, ow_ref[...],
                            preferred_element_type=jnp.float32)
                    + ob_ref[...])


def _stage3(pooled, value_W, value_b, out_W, out_b):
    return pl.pallas_call(
        _stage3_body,
        out_shape=jax.ShapeDtypeStruct((S_PAD, D), jnp.float32),
    )(pooled, value_W, value_b.reshape(1, D),
      out_W, out_b.reshape(1, D))


def kernel(nodes, indices, key_W, key_b, gate_W, gate_b, query_W, query_b,
           value_W, value_b, out_W, out_b):
    idx32 = indices.astype(jnp.int32)
    bounds = _stage0(idx32).reshape(128)
    comb = _stage1(nodes, key_W, key_b, gate_W, gate_b, query_W, query_b)
    pooled = _stage2(comb, idx32, bounds).reshape(S_PAD, COLS)
    out_pad = _stage3(pooled, value_W, value_b, out_W, out_b)
    return out_pad[:S]
